# Initial kernel scaffold; baseline (speedup 1.0000x reference)
import functools

import jax
import jax.numpy as jnp
from jax import lax
from jax.experimental import pallas as pl
from jax.experimental.pallas import tpu as pltpu
from jax.experimental.pallas import tpu_sc as plsc

N = 10000
C = 128
E = 320000
EPS = 1e-5

NC = 2
NS = 16
NW = NC * NS
CHUNK = 128
E_PAD = ((E + NW * CHUNK - 1) // (NW * CHUNK)) * (NW * CHUNK)
EPW = E_PAD // NW
NCH = EPW // CHUNK
ROWS_T = N // NS
DW = 16


def _mesh():
    return plsc.VectorSubcoreMesh(
        core_axis_name="c", subcore_axis_name="s", num_cores=NC, num_subcores=NS
    )


def _deg_body(dst_hbm, ones_hbm, zeros_hbm, out_hbm, didx, ones_v, xfer, deg_sh):
    c = lax.axis_index("c")
    s = lax.axis_index("s")
    w = c * NS + s
    base = w * EPW
    r0 = s * ROWS_T
    pltpu.sync_copy(ones_hbm, ones_v)
    pltpu.sync_copy(zeros_hbm, xfer)
    pltpu.sync_copy(xfer, deg_sh.at[pl.ds(r0, ROWS_T)])
    plsc.subcore_barrier()

    def step(i, carry):
        off = base + i * CHUNK
        pltpu.sync_copy(dst_hbm.at[pl.ds(off, CHUNK)], didx)
        pltpu.sync_copy(ones_v, deg_sh.at[didx], add=True)
        return carry

    lax.fori_loop(0, NCH, step, 0)
    plsc.subcore_barrier()
    pltpu.sync_copy(deg_sh.at[pl.ds(r0, ROWS_T)], xfer)
    pltpu.sync_copy(xfer, out_hbm.at[pl.ds(c * N + r0, ROWS_T)])


def _deg_call(dst_p, ones, zeros):
    return pl.kernel(
        _deg_body,
        out_type=jax.ShapeDtypeStruct((NC * N, DW), jnp.float32),
        mesh=_mesh(),
        scratch_types=[
            pltpu.VMEM((CHUNK,), jnp.int32),
            pltpu.VMEM((CHUNK, DW), jnp.float32),
            pltpu.VMEM((ROWS_T, DW), jnp.float32),
            pltpu.VMEM_SHARED((N + 8, DW), jnp.float32),
        ],
    )(dst_p, ones, zeros)


def _msg_body(h2_hbm, src_hbm, dst_hbm, out_hbm, sidx, didx, rows, xfer, acc_sh, sem):
    c = lax.axis_index("c")
    s = lax.axis_index("s")
    w = c * NS + s
    base = w * EPW
    r0 = s * ROWS_T
    pltpu.sync_copy(h2_hbm.at[pl.ds(r0, ROWS_T)], xfer)
    pltpu.sync_copy(xfer, acc_sh.at[pl.ds(r0, ROWS_T)])
    plsc.subcore_barrier()

    def step(i, carry):
        off = base + i * CHUNK
        pltpu.sync_copy(src_hbm.at[pl.ds(off, CHUNK)], sidx)
        pltpu.sync_copy(dst_hbm.at[pl.ds(off, CHUNK)], didx)
        pltpu.async_copy(h2_hbm.at[sidx], rows, sem).wait()
        pltpu.sync_copy(rows, acc_sh.at[didx], add=True)
        return carry

    lax.fori_loop(0, NCH, step, 0)
    plsc.subcore_barrier()
    pltpu.sync_copy(acc_sh.at[pl.ds(r0, ROWS_T)], xfer)
    pltpu.sync_copy(xfer, out_hbm.at[pl.ds(c * N + r0, ROWS_T)])


def _msg_call(h2, src_p, dst_p):
    return pl.kernel(
        _msg_body,
        out_type=jax.ShapeDtypeStruct((NC * N, C), jnp.float32),
        mesh=_mesh(),
        scratch_types=[
            pltpu.VMEM((CHUNK,), jnp.int32),
            pltpu.VMEM((CHUNK,), jnp.int32),
            pltpu.VMEM((CHUNK, C), jnp.float32),
            pltpu.VMEM((ROWS_T, C), jnp.float32),
            pltpu.VMEM_SHARED((N + 8, C), jnp.float32),
            pltpu.SemaphoreType.DMA,
        ],
    )(h2, src_p, dst_p)


def _dense_body(x_ref, g0_ref, b0_ref, w_ref, degp_ref, h2_ref):
    x = x_ref[...]
    mean = jnp.mean(x, axis=0, keepdims=True)
    xc = x - mean
    var = jnp.mean(xc * xc, axis=0, keepdims=True)
    xn = xc * lax.rsqrt(var + EPS) * g0_ref[...] + b0_ref[...]
    h = jnp.dot(xn, w_ref[...], preferred_element_type=jnp.float32)
    degp = degp_ref[...]
    deg = 1.0 + degp[:N, 0:1] + degp[N:, 0:1]
    h2_ref[...] = h * lax.rsqrt(deg)


def _dense_call(x, g0, b0, W, degp):
    return pl.pallas_call(
        _dense_body,
        out_shape=jax.ShapeDtypeStruct((N, C), jnp.float32),
    )(x, g0, b0, W, degp)


def _finish_body(acc_ref, h2_ref, degp_ref, b_ref, g1_ref, b1_ref, y_ref):
    degp = degp_ref[...]
    deg = 1.0 + degp[:N, 0:1] + degp[N:, 0:1]
    acc = acc_ref[:N, :] + acc_ref[N:, :]
    pre = (acc - h2_ref[...]) * lax.rsqrt(deg) + b_ref[...]
    r = jnp.maximum(pre, 0.0)
    mean = jnp.mean(r, axis=0, keepdims=True)
    rc = r - mean
    var = jnp.mean(rc * rc, axis=0, keepdims=True)
    y_ref[...] = rc * lax.rsqrt(var + EPS) * g1_ref[...] + b1_ref[...]


def _finish_call(accp, h2, degp, b, g1, b1):
    return pl.pallas_call(
        _finish_body,
        out_shape=jax.ShapeDtypeStruct((N, C), jnp.float32),
    )(accp, h2, degp, b, g1, b1)


@jax.jit
def kernel(x, edge_index, bn0_gamma, bn0_beta, W, b, bn1_gamma, bn1_beta):
    ei = edge_index.astype(jnp.int32)
    pad = E_PAD - E
    src_p = jnp.concatenate([ei[0], jnp.zeros((pad,), jnp.int32)])
    dst_p = jnp.concatenate([ei[1], jnp.full((pad,), N, jnp.int32)])

    ones = jnp.ones((CHUNK, DW), jnp.float32)
    zeros = jnp.zeros((ROWS_T, DW), jnp.float32)

    degp = _deg_call(dst_p, ones, zeros)
    h2 = _dense_call(
        x, bn0_gamma.reshape(1, C), bn0_beta.reshape(1, C), W, degp
    )
    accp = _msg_call(h2, src_p, dst_p)
    y = _finish_call(
        accp, h2, degp, b.reshape(1, C),
        bn1_gamma.reshape(1, C), bn1_beta.reshape(1, C),
    )
    return y


# speedup vs baseline: 13.2241x; 13.2241x over previous
import functools

import jax
import jax.numpy as jnp
from jax import lax
from jax.experimental import pallas as pl
from jax.experimental.pallas import tpu as pltpu
from jax.experimental.pallas import tpu_sc as plsc

N = 10000
C = 128
E = 320000
EPS = 1e-5

NC = 2
NS = 16
NW = NC * NS
CHUNK = 128
E_PAD = ((E + NW * CHUNK - 1) // (NW * CHUNK)) * (NW * CHUNK)
EPW = E_PAD // NW
NCH = EPW // CHUNK
N_PAD = 10240
ROWS_T = N_PAD // NS
DW = 128


def _mesh():
    return plsc.VectorSubcoreMesh(
        core_axis_name="c", subcore_axis_name="s", num_cores=NC, num_subcores=NS
    )


def _deg_body(dst_hbm, ones_hbm, zeros_hbm, out_hbm, didx, ones_v, deg_sh):
    c = lax.axis_index("c")
    s = lax.axis_index("s")
    w = c * NS + s
    base = w * EPW
    r0 = s * ROWS_T
    pltpu.sync_copy(ones_hbm, ones_v)
    pltpu.sync_copy(zeros_hbm, deg_sh.at[pl.ds(r0, ROWS_T)])
    plsc.subcore_barrier()

    def step(i, carry):
        off = base + i * CHUNK
        pltpu.sync_copy(dst_hbm.at[pl.ds(off, CHUNK)], didx)
        pltpu.sync_copy(ones_v, deg_sh.at[didx], add=True)
        return carry

    lax.fori_loop(0, NCH, step, 0)
    plsc.subcore_barrier()
    pltpu.sync_copy(deg_sh.at[pl.ds(r0, ROWS_T)], out_hbm.at[pl.ds(c * N_PAD + r0, ROWS_T)])


def _deg_call(dst_p, ones, zeros):
    return pl.kernel(
        _deg_body,
        out_type=jax.ShapeDtypeStruct((NC * N_PAD, DW), jnp.float32),
        mesh=_mesh(),
        scratch_types=[
            pltpu.VMEM((CHUNK,), jnp.int32),
            pltpu.VMEM((CHUNK, DW), jnp.float32),
            pltpu.VMEM_SHARED((N_PAD, DW), jnp.float32),
        ],
    )(dst_p, ones, zeros)


def _msg_body(h2_hbm, src_hbm, dst_hbm, out_hbm, sidx, didx, rows, acc_sh, sem):
    c = lax.axis_index("c")
    s = lax.axis_index("s")
    w = c * NS + s
    base = w * EPW
    r0 = s * ROWS_T
    pltpu.sync_copy(h2_hbm.at[pl.ds(r0, ROWS_T)], acc_sh.at[pl.ds(r0, ROWS_T)])
    plsc.subcore_barrier()

    def step(i, carry):
        off = base + i * CHUNK
        pltpu.sync_copy(src_hbm.at[pl.ds(off, CHUNK)], sidx)
        pltpu.sync_copy(dst_hbm.at[pl.ds(off, CHUNK)], didx)
        pltpu.async_copy(h2_hbm.at[sidx], rows, sem).wait()
        pltpu.sync_copy(rows, acc_sh.at[didx], add=True)
        return carry

    lax.fori_loop(0, NCH, step, 0)
    plsc.subcore_barrier()
    pltpu.sync_copy(acc_sh.at[pl.ds(r0, ROWS_T)], out_hbm.at[pl.ds(c * N_PAD + r0, ROWS_T)])


def _msg_call(h2, src_p, dst_p):
    return pl.kernel(
        _msg_body,
        out_type=jax.ShapeDtypeStruct((NC * N_PAD, C), jnp.float32),
        mesh=_mesh(),
        scratch_types=[
            pltpu.VMEM((CHUNK,), jnp.int32),
            pltpu.VMEM((CHUNK,), jnp.int32),
            pltpu.VMEM((CHUNK, C), jnp.float32),
            pltpu.VMEM_SHARED((N_PAD, C), jnp.float32),
            pltpu.SemaphoreType.DMA,
        ],
    )(h2, src_p, dst_p)


def _dense_body(x_ref, g0_ref, b0_ref, w_ref, degp_ref, h2_ref):
    x = x_ref[...]
    mean = jnp.mean(x, axis=0, keepdims=True)
    xc = x - mean
    var = jnp.mean(xc * xc, axis=0, keepdims=True)
    xn = xc * lax.rsqrt(var + EPS) * g0_ref[...] + b0_ref[...]
    h = jnp.dot(xn, w_ref[...], preferred_element_type=jnp.float32)
    degp = degp_ref[...]
    deg = 1.0 + degp[:N, 0:1] + degp[N_PAD:N_PAD + N, 0:1]
    h2_ref[:N, :] = h * lax.rsqrt(deg)
    h2_ref[N:, :] = jnp.zeros((N_PAD - N, C), jnp.float32)


def _dense_call(x, g0, b0, W, degp):
    return pl.pallas_call(
        _dense_body,
        out_shape=jax.ShapeDtypeStruct((N_PAD, C), jnp.float32),
    )(x, g0, b0, W, degp)


def _finish_body(acc_ref, h2_ref, degp_ref, b_ref, g1_ref, b1_ref, y_ref):
    degp = degp_ref[...]
    deg = 1.0 + degp[:N, 0:1] + degp[N_PAD:N_PAD + N, 0:1]
    acc = acc_ref[:N, :] + acc_ref[N_PAD:N_PAD + N, :]
    pre = (acc - h2_ref[:N, :]) * lax.rsqrt(deg) + b_ref[...]
    r = jnp.maximum(pre, 0.0)
    mean = jnp.mean(r, axis=0, keepdims=True)
    rc = r - mean
    var = jnp.mean(rc * rc, axis=0, keepdims=True)
    y_ref[...] = rc * lax.rsqrt(var + EPS) * g1_ref[...] + b1_ref[...]


def _finish_call(accp, h2, degp, b, g1, b1):
    return pl.pallas_call(
        _finish_body,
        out_shape=jax.ShapeDtypeStruct((N, C), jnp.float32),
    )(accp, h2, degp, b, g1, b1)


@jax.jit
def kernel(x, edge_index, bn0_gamma, bn0_beta, W, b, bn1_gamma, bn1_beta):
    ei = edge_index.astype(jnp.int32)
    pad = E_PAD - E
    src_p = jnp.concatenate([ei[0], jnp.zeros((pad,), jnp.int32)])
    dst_p = jnp.concatenate([ei[1], jnp.full((pad,), N, jnp.int32)])

    ones = jnp.ones((CHUNK, DW), jnp.float32)
    zeros = jnp.zeros((ROWS_T, DW), jnp.float32)

    degp = _deg_call(dst_p, ones, zeros)
    h2 = _dense_call(
        x, bn0_gamma.reshape(1, C), bn0_beta.reshape(1, C), W, degp
    )
    accp = _msg_call(h2, src_p, dst_p)
    y = _finish_call(
        accp, h2, degp, b.reshape(1, C),
        bn1_gamma.reshape(1, C), bn1_beta.reshape(1, C),
    )
    return y


# speedup vs baseline: 28.5947x; 2.1623x over previous
import functools

import jax
import jax.numpy as jnp
from jax import lax
from jax.experimental import pallas as pl
from jax.experimental.pallas import tpu as pltpu
from jax.experimental.pallas import tpu_sc as plsc

N = 10000
C = 128
E = 320000
EPS = 1e-5

NC = 2
NS = 16
NW = NC * NS
CHUNK = 128
NCH = 80
HALF = NCH // 2
E_PAD = NW * NCH * CHUNK
EPW = E_PAD // NW
N_PAD = 10240
ROWS_T = N_PAD // NS
DW = 128


def _mesh():
    return plsc.VectorSubcoreMesh(
        core_axis_name="c", subcore_axis_name="s", num_cores=NC, num_subcores=NS
    )


def _deg_body(dst_hbm, ones_hbm, zeros_hbm, out_hbm, didx_all,
              ones_v, deg_sh, ssem0, ssem1):
    c = lax.axis_index("c")
    s = lax.axis_index("s")
    w = c * NS + s
    r0 = s * ROWS_T
    pltpu.sync_copy(ones_hbm, ones_v)
    pltpu.sync_copy(dst_hbm.at[pl.ds(w * NCH, NCH)], didx_all)
    pltpu.sync_copy(zeros_hbm, deg_sh.at[pl.ds(r0, ROWS_T)])
    plsc.subcore_barrier()

    def step(k, carry):
        i = 2 * k

        @pl.when(k > 0)
        def _():
            pltpu.make_async_copy(ones_v, deg_sh.at[didx_all.at[i - 2]], ssem0).wait()
            pltpu.make_async_copy(ones_v, deg_sh.at[didx_all.at[i - 1]], ssem1).wait()

        pltpu.async_copy(ones_v, deg_sh.at[didx_all.at[i]], ssem0, add=True)
        pltpu.async_copy(ones_v, deg_sh.at[didx_all.at[i + 1]], ssem1, add=True)
        return carry

    lax.fori_loop(0, NCH // 2, step, 0)
    pltpu.make_async_copy(ones_v, deg_sh.at[didx_all.at[0]], ssem0).wait()
    pltpu.make_async_copy(ones_v, deg_sh.at[didx_all.at[1]], ssem1).wait()
    plsc.subcore_barrier()
    pltpu.sync_copy(deg_sh.at[pl.ds(r0, ROWS_T)], out_hbm.at[pl.ds(c * N_PAD + r0, ROWS_T)])


def _deg_call(dst2d, ones, zeros):
    return pl.kernel(
        _deg_body,
        out_type=jax.ShapeDtypeStruct((NC * N_PAD, DW), jnp.float32),
        mesh=_mesh(),
        scratch_types=[
            pltpu.VMEM((NCH, CHUNK), jnp.int32),
            pltpu.VMEM((CHUNK, DW), jnp.float32),
            pltpu.VMEM_SHARED((N_PAD, DW), jnp.float32),
            pltpu.SemaphoreType.DMA,
            pltpu.SemaphoreType.DMA,
        ],
    )(dst2d, ones, zeros)


def _msg_body(h2_hbm, src2d_hbm, dst2d_hbm, out_hbm, sidx_h, didx_h,
              rows0, rows1, acc_sh, gsem0, gsem1, ssem0, ssem1):
    c = lax.axis_index("c")
    s = lax.axis_index("s")
    w = c * NS + s
    r0 = s * ROWS_T
    pltpu.sync_copy(h2_hbm.at[pl.ds(r0, ROWS_T)], acc_sh.at[pl.ds(r0, ROWS_T)])
    plsc.subcore_barrier()

    for h in range(2):
        base_row = w * NCH + h * HALF
        pltpu.sync_copy(src2d_hbm.at[pl.ds(base_row, HALF)], sidx_h)
        pltpu.sync_copy(dst2d_hbm.at[pl.ds(base_row, HALF)], didx_h)

        pltpu.async_copy(h2_hbm.at[sidx_h.at[0]], rows0, gsem0)

        def step(k, carry):
            i = 2 * k

            @pl.when(i > 0)
            def _():
                pltpu.make_async_copy(rows1, acc_sh.at[didx_h.at[i - 1]], ssem1).wait()

            g1 = pltpu.async_copy(h2_hbm.at[sidx_h.at[i + 1]], rows1, gsem1)

            pltpu.make_async_copy(h2_hbm.at[sidx_h.at[i]], rows0, gsem0).wait()
            s0 = pltpu.async_copy(rows0, acc_sh.at[didx_h.at[i]], ssem0, add=True)

            @pl.when(i + 2 < HALF)
            def _():
                s0.wait()
                pltpu.async_copy(h2_hbm.at[sidx_h.at[i + 2]], rows0, gsem0)

            g1.wait()
            pltpu.async_copy(rows1, acc_sh.at[didx_h.at[i + 1]], ssem1, add=True)
            return carry

        lax.fori_loop(0, HALF // 2, step, 0)
        pltpu.make_async_copy(rows0, acc_sh.at[didx_h.at[0]], ssem0).wait()
        pltpu.make_async_copy(rows1, acc_sh.at[didx_h.at[1]], ssem1).wait()

    plsc.subcore_barrier()
    pltpu.sync_copy(acc_sh.at[pl.ds(r0, ROWS_T)], out_hbm.at[pl.ds(c * N_PAD + r0, ROWS_T)])


def _msg_call(h2, src2d, dst2d):
    return pl.kernel(
        _msg_body,
        out_type=jax.ShapeDtypeStruct((NC * N_PAD, C), jnp.float32),
        mesh=_mesh(),
        scratch_types=[
            pltpu.VMEM((HALF, CHUNK), jnp.int32),
            pltpu.VMEM((HALF, CHUNK), jnp.int32),
            pltpu.VMEM((CHUNK, C), jnp.float32),
            pltpu.VMEM((CHUNK, C), jnp.float32),
            pltpu.VMEM_SHARED((N_PAD, C), jnp.float32),
            pltpu.SemaphoreType.DMA,
            pltpu.SemaphoreType.DMA,
            pltpu.SemaphoreType.DMA,
            pltpu.SemaphoreType.DMA,
        ],
    )(h2, src2d, dst2d)


def _mm_body(x_ref, g0_ref, b0_ref, w_ref, h_ref):
    x = x_ref[...]
    mean = jnp.mean(x, axis=0, keepdims=True)
    xc = x - mean
    var = jnp.mean(xc * xc, axis=0, keepdims=True)
    xn = xc * lax.rsqrt(var + EPS) * g0_ref[...] + b0_ref[...]
    h_ref[:N, :] = jnp.dot(xn, w_ref[...], preferred_element_type=jnp.float32)
    h_ref[N:, :] = jnp.zeros((N_PAD - N, C), jnp.float32)


def _mm_call(x, g0, b0, W):
    return pl.pallas_call(
        _mm_body,
        out_shape=jax.ShapeDtypeStruct((N_PAD, C), jnp.float32),
    )(x, g0, b0, W)


def _scale_body(h_ref, degp_ref, h2_ref):
    degp = degp_ref[...]
    deg = 1.0 + degp[:N_PAD, 0:1] + degp[N_PAD:, 0:1]
    h2_ref[...] = h_ref[...] * lax.rsqrt(deg)


def _scale_call(h, degp):
    return pl.pallas_call(
        _scale_body,
        out_shape=jax.ShapeDtypeStruct((N_PAD, C), jnp.float32),
    )(h, degp)


def _finish_body(acc_ref, h2_ref, degp_ref, b_ref, g1_ref, b1_ref, y_ref):
    degp = degp_ref[...].astype(jnp.float32)
    deg = 1.0 + degp[:N, 0:1] + degp[N_PAD:N_PAD + N, 0:1]
    acc = acc_ref[:N, :] + acc_ref[N_PAD:N_PAD + N, :]
    pre = (acc - h2_ref[:N, :]) * lax.rsqrt(deg) + b_ref[...]
    r = jnp.maximum(pre, 0.0)
    mean = jnp.mean(r, axis=0, keepdims=True)
    rc = r - mean
    var = jnp.mean(rc * rc, axis=0, keepdims=True)
    y_ref[...] = rc * lax.rsqrt(var + EPS) * g1_ref[...] + b1_ref[...]


def _finish_call(accp, h2, degp, b, g1, b1):
    return pl.pallas_call(
        _finish_body,
        out_shape=jax.ShapeDtypeStruct((N, C), jnp.float32),
    )(accp, h2, degp, b, g1, b1)


@jax.jit
def kernel(x, edge_index, bn0_gamma, bn0_beta, W, b, bn1_gamma, bn1_beta):
    ei = edge_index.astype(jnp.int32)
    pad = E_PAD - E
    pad_src = (jnp.arange(pad, dtype=jnp.int32) * 131) % N
    pad_dst = N + (jnp.arange(pad, dtype=jnp.int32) % (N_PAD - N))
    src2d = jnp.concatenate([ei[0], pad_src]).reshape(-1, CHUNK)
    dst2d = jnp.concatenate([ei[1], pad_dst]).reshape(-1, CHUNK)

    ones = jnp.ones((CHUNK, DW), jnp.float32)
    zeros = jnp.zeros((ROWS_T, DW), jnp.float32)

    degp = _deg_call(dst2d, ones, zeros)
    h = _mm_call(
        x, bn0_gamma.reshape(1, C), bn0_beta.reshape(1, C), W
    )
    h2 = _scale_call(h, degp)
    accp = _msg_call(h2, src2d, dst2d)
    y = _finish_call(
        accp, h2, degp, b.reshape(1, C),
        bn1_gamma.reshape(1, C), bn1_beta.reshape(1, C),
    )
    return y
